# baseline (speedup 1.0000x reference)
import functools

import jax
import jax.numpy as jnp
from jax import lax
from jax.experimental import pallas as pl
from jax.experimental.pallas import tpu as pltpu
from jax.experimental.pallas import tpu_sc as plsc

ATTRS_ = (25, 6, 18, 3, 9, 6, 4, 5, 5, 3, 3, 3, 3, 3, 10)
DIMS_ = (10, 3, 9, 3, 5, 3, 2, 3, 3, 2, 2, 2, 2, 2, 5)
B_ = 16384
OUT_W = 1 + sum(DIMS_)

NW = 32
ROWS_PER_W = B_ // NW
L = 16
NBLK = ROWS_PER_W // L


def _body(xt_hbm, *refs):
    tt_hbm = refs[:15]
    out_hbm = refs[15]
    xt_v = refs[16]
    tt_v = refs[17:32]
    out_v = refs[32]
    sem = refs[33]
    sem2 = refs[34]
    sem3 = refs[35]

    wid = lax.axis_index("s") * 2 + lax.axis_index("c")
    base = wid * ROWS_PER_W
    H = ROWS_PER_W // 2
    copies = [pltpu.async_copy(tt_hbm[i], tt_v[i], sem) for i in range(15)]
    copies.append(
        pltpu.async_copy(
            xt_hbm.at[:, pl.ds(base, H)], xt_v.at[:, pl.ds(0, H)], sem
        )
    )
    x2 = pltpu.async_copy(
        xt_hbm.at[:, pl.ds(base + H, H)], xt_v.at[:, pl.ds(H, H)], sem2
    )

    dsplat = [jnp.full((L,), d, jnp.int32) for d in range(max(DIMS_))]

    def block(b, _):
        rr = b * L
        sl = pl.ds(rr, L)
        out_v[0, sl] = xt_v[0, sl]
        ridx = [None] * 16
        for j in range(1, 16):
            ridx[j] = xt_v[j, sl].astype(jnp.int32)
        c = 1
        for i in range(15):
            for d in range(DIMS_[i]):
                if i == 0:
                    v = plsc.load_gather(tt_v[0], [ridx[1], dsplat[d]])
                else:
                    v = plsc.load_gather(tt_v[i], [dsplat[d], ridx[i + 1]])
                out_v[c, sl] = v
                c += 1
        return _

    for c in copies:
        c.wait()
    lax.fori_loop(0, NBLK // 2, block, None)
    o1 = pltpu.async_copy(
        out_v.at[:, pl.ds(0, H)], out_hbm.at[:, pl.ds(base, H)], sem3
    )
    x2.wait()
    lax.fori_loop(NBLK // 2, NBLK, block, None)
    o2 = pltpu.async_copy(
        out_v.at[:, pl.ds(H, H)], out_hbm.at[:, pl.ds(base + H, H)], sem3
    )
    o1.wait()
    o2.wait()


@functools.partial(jax.jit, static_argnames=("interpret",))
def kernel(x, tables, interpret=False):
    run = pl.kernel(
        _body,
        out_type=jax.ShapeDtypeStruct((OUT_W, B_), jnp.float32),
        mesh=plsc.VectorSubcoreMesh(
            core_axis_name="c", subcore_axis_name="s",
            num_cores=2, num_subcores=16,
        ),
        scratch_types=[
            pltpu.VMEM((16, ROWS_PER_W), jnp.float32),
            *[
                pltpu.VMEM(
                    (ATTRS_[i], DIMS_[i]) if i == 0 else (DIMS_[i], ATTRS_[i]),
                    jnp.float32,
                )
                for i in range(15)
            ],
            pltpu.VMEM((OUT_W, ROWS_PER_W), jnp.float32),
            pltpu.SemaphoreType.DMA,
            pltpu.SemaphoreType.DMA,
            pltpu.SemaphoreType.DMA,
        ],
        compiler_params=pltpu.CompilerParams(
            needs_layout_passes=False, use_tc_tiling_on_sc=True
        ),
        interpret=interpret,
    )
    return run(
        x.T, tables[0], *[t.T for t in tables[1:]]
    ).T


# speedup vs baseline: 1.0988x; 1.0988x over previous
import functools

import jax
import jax.numpy as jnp
from jax import lax
from jax.experimental import pallas as pl
from jax.experimental.pallas import tpu as pltpu
from jax.experimental.pallas import tpu_sc as plsc

ATTRS_ = (25, 6, 18, 3, 9, 6, 4, 5, 5, 3, 3, 3, 3, 3, 10)
DIMS_ = (10, 3, 9, 3, 5, 3, 2, 3, 3, 2, 2, 2, 2, 2, 5)
B_ = 16384
OUT_W = 1 + sum(DIMS_)

NW = 32
ROWS_PER_W = B_ // NW
L = 16
NBLK = ROWS_PER_W // L


def _body(xt_hbm, *refs):
    tt_hbm = refs[:15]
    out_hbm = refs[15]
    xt_v = refs[16]
    tt_v = refs[17:32]
    out_v = refs[32]
    sem = refs[33]
    sem2 = refs[34]
    sem3 = refs[35]

    wid = lax.axis_index("s") * 2 + lax.axis_index("c")
    base = wid * ROWS_PER_W
    H = ROWS_PER_W // 2
    copies = [pltpu.async_copy(tt_hbm[i], tt_v[i], sem) for i in range(15)]
    copies.append(
        pltpu.async_copy(
            xt_hbm.at[:, pl.ds(base, H)], xt_v.at[:, pl.ds(0, H)], sem
        )
    )
    x2 = pltpu.async_copy(
        xt_hbm.at[:, pl.ds(base + H, H)], xt_v.at[:, pl.ds(H, H)], sem2
    )

    dsplat = [jnp.full((L,), d, jnp.int32) for d in range(max(DIMS_))]

    def block(b, _):
        rr = b * L
        sl = pl.ds(rr, L)
        out_v[0, sl] = xt_v[0, sl]
        ridx = [None] * 16
        for j in range(1, 16):
            ridx[j] = xt_v[j, sl].astype(jnp.int32)
        c = 1
        for i in range(15):
            for d in range(DIMS_[i]):
                out_v[c, sl] = plsc.load_gather(
                    tt_v[i], [dsplat[d], ridx[i + 1]]
                )
                c += 1
        return _

    for c in copies:
        c.wait()
    lax.fori_loop(0, NBLK // 2, block, None)
    o1 = pltpu.async_copy(
        out_v.at[:, pl.ds(0, H)], out_hbm.at[:, pl.ds(base, H)], sem3
    )
    x2.wait()
    lax.fori_loop(NBLK // 2, NBLK, block, None)
    o2 = pltpu.async_copy(
        out_v.at[:, pl.ds(H, H)], out_hbm.at[:, pl.ds(base + H, H)], sem3
    )
    o1.wait()
    o2.wait()


@functools.partial(jax.jit, static_argnames=("interpret",))
def kernel(x, tables, interpret=False):
    run = pl.kernel(
        _body,
        out_type=jax.ShapeDtypeStruct((OUT_W, B_), jnp.float32),
        mesh=plsc.VectorSubcoreMesh(
            core_axis_name="c", subcore_axis_name="s",
            num_cores=2, num_subcores=16,
        ),
        scratch_types=[
            pltpu.VMEM((16, ROWS_PER_W), jnp.float32),
            *[
                pltpu.VMEM((DIMS_[i], ATTRS_[i]), jnp.float32)
                for i in range(15)
            ],
            pltpu.VMEM((OUT_W, ROWS_PER_W), jnp.float32),
            pltpu.SemaphoreType.DMA,
            pltpu.SemaphoreType.DMA,
            pltpu.SemaphoreType.DMA,
        ],
        compiler_params=pltpu.CompilerParams(
            needs_layout_passes=False, use_tc_tiling_on_sc=True
        ),
        interpret=interpret,
    )
    return run(x.T, *[t.T for t in tables]).T
